# baseline (speedup 1.0000x reference)
import jax
import jax.numpy as jnp
from jax.experimental import pallas as pl
from jax.experimental.pallas import tpu as pltpu

VOCAB = 17
EMB_DIM = 32
CONTEXT = 2
IN_DIM = EMB_DIM * CONTEXT


def _body(x_ref, emb_ref, w_ref, b_ref, out_ref):
    e0 = emb_ref[pl.ds(x_ref[0], 1), :]
    e1 = emb_ref[pl.ds(x_ref[1], 1), :]
    e = jnp.concatenate([e0, e1], axis=1)
    y = jax.lax.dot_general(
        e.astype(jnp.bfloat16), w_ref[...].astype(jnp.bfloat16),
        dimension_numbers=(((1,), (1,)), ((), ())),
        preferred_element_type=jnp.float32,
    )
    out_ref[...] = y + b_ref[...]


def kernel(x, emb, W, b):
    return pl.pallas_call(
        _body,
        out_shape=jax.ShapeDtypeStruct((1, VOCAB), jnp.float32),
        in_specs=[
            pl.BlockSpec(memory_space=pltpu.SMEM),
            pl.BlockSpec(memory_space=pltpu.VMEM),
            pl.BlockSpec(memory_space=pltpu.VMEM),
            pl.BlockSpec(memory_space=pltpu.VMEM),
        ],
        out_specs=pl.BlockSpec(memory_space=pltpu.VMEM),
    )(x.astype(jnp.int32), emb, W, b.reshape(1, VOCAB))


# speedup vs baseline: 1.0009x; 1.0009x over previous
import jax
import jax.numpy as jnp
from jax.experimental import pallas as pl
from jax.experimental.pallas import tpu as pltpu

VOCAB = 17
EMB_DIM = 32
CONTEXT = 2
IN_DIM = EMB_DIM * CONTEXT


def _body(x_ref, emb_ref, w_ref, b_ref, out_ref):
    e0 = emb_ref[pl.ds(x_ref[0], 1), :]
    e1 = emb_ref[pl.ds(x_ref[1], 1), :]
    e = jnp.concatenate([e0, e1], axis=1)
    y = jax.lax.dot_general(
        e, w_ref[...],
        dimension_numbers=(((1,), (1,)), ((), ())),
        preferred_element_type=jnp.float32,
    )
    out_ref[...] = y + b_ref[...].reshape(1, VOCAB)


def kernel(x, emb, W, b):
    return pl.pallas_call(
        _body,
        out_shape=jax.ShapeDtypeStruct((1, VOCAB), jnp.float32),
        in_specs=[
            pl.BlockSpec(memory_space=pltpu.SMEM),
            pl.BlockSpec(memory_space=pltpu.VMEM),
            pl.BlockSpec(memory_space=pltpu.VMEM),
            pl.BlockSpec(memory_space=pltpu.VMEM),
        ],
        out_specs=pl.BlockSpec(memory_space=pltpu.VMEM),
    )(x.astype(jnp.int32), emb, W, b)
